# Initial kernel scaffold; baseline (speedup 1.0000x reference)
import jax
import jax.numpy as jnp
from jax.experimental import pallas as pl


def kernel(node_attrs, vectors, senders, receivers, W_emb1, W_emb2, W_v, W0a, W0b, Wg0, W1a, W1b, Wg1, W_out):
    raise NotImplementedError("write your pallas kernel here")


# speedup vs baseline: 8.9044x; 8.9044x over previous
import functools

import numpy as np
import jax
import jax.numpy as jnp
from jax import lax
from jax.experimental import pallas as pl
from jax.experimental.pallas import tpu as pltpu

E = 160000
NN = 10000
EP = 163840
NE = 10240
D = 128
DV = 16
NR = 8
BE = 2048
RSQRT2 = float(1.0 / np.sqrt(2.0))
SQRT2 = float(np.sqrt(2.0))
PI = float(np.pi)

_INTERPRET = False


def _expand_mats():
    R = np.zeros((16, 256), np.float32)
    T = np.zeros((16, 256), np.float32)
    S = np.zeros((256, 16), np.float32)
    for i in range(16):
        for c in range(16):
            R[i, i * 16 + c] = 1.0
            T[c, i * 16 + c] = 1.0
            S[i * 16 + c, c] = 1.0
    return R, T, S


_R_NP, _T_NP, _S_NP = _expand_mats()


def _dot(a, b):
    return jnp.dot(a, b, preferred_element_type=jnp.float32)


def _silu(x):
    return x * jax.nn.sigmoid(x)


def _node_mm_body(x_ref, ws_ref, wr_ref, as_ref, ar_ref):
    x = x_ref[...]
    as_ref[...] = _dot(x, ws_ref[...])
    ar_ref[...] = _dot(x, wr_ref[...])


def _edge1_body(v4_ref, g_ref, w1rb_ref, wemb2_ref, wv_ref, r_ref, t_ref,
                cut_ref, x0_ref, xv_ref, sh_ref, v0_ref):
    v = v4_ref[...]
    d = jnp.sqrt(jnp.sum(v * v, axis=1, keepdims=True))
    xb = jnp.clip(d, 1e-6, 1.0)
    ns = lax.broadcasted_iota(jnp.int32, (v.shape[0], NR), 1).astype(jnp.float32) + 1.0
    rb = SQRT2 * jnp.sin(ns * (PI * xb)) / xb
    xe = jnp.clip(d, 0.0, 1.0)
    x2 = xe * xe
    x3 = x2 * xe
    x6 = x3 * x3
    cut = 1.0 - 28.0 * x6 + 48.0 * x6 * xe - 21.0 * x6 * x2
    u = v / jnp.maximum(d, 1e-6)
    ux, uy, uz = u[:, 0:1], u[:, 1:2], u[:, 2:3]
    one = jnp.ones_like(ux)
    sh = jnp.concatenate([
        one, ux, uy, uz, ux * uy, uy * uz, 3.0 * uz * uz - 1.0, ux * uz,
        ux * ux - uy * uy, uy * (3.0 * ux * ux - uy * uy), ux * uy * uz,
        uy * (5.0 * uz * uz - 1.0), uz * (5.0 * uz * uz - 3.0),
        ux * (5.0 * uz * uz - 1.0), uz * (ux * ux - uy * uy),
        ux * (ux * ux - 3.0 * uy * uy)], axis=1)
    h = g_ref[...] + _dot(rb, w1rb_ref[...])
    x0 = _silu(h)
    x0 = _dot(x0, wemb2_ref[...]) * cut
    xv = _dot(x0, wv_ref[...])
    v0 = _dot(sh, r_ref[...]) * _dot(xv, t_ref[...])
    cut_ref[...] = cut
    x0_ref[...] = x0
    xv_ref[...] = xv
    sh_ref[...] = sh
    v0_ref[...] = v0


def _edge2_body(x0_ref, xv_ref, sh_ref, venv_ref, cut_ref,
                wax_ref, wat_ref, wb_ref, wg_ref, r_ref, s_ref,
                x1_ref, g0_ref, s1_ref):
    venv = venv_ref[...]
    cut = cut_ref[...]
    x0 = x0_ref[...]
    shr = _dot(sh_ref[...], r_ref[...])
    w_raw = _dot(shr * venv, s_ref[...])
    t0 = xv_ref[...] * w_raw * (1.0 / 16.0)
    pre = _dot(x0, wax_ref[...]) + _dot(t0, wat_ref[...])
    x1 = (_dot(_silu(pre), wb_ref[...]) * cut + x0) * RSQRT2
    g0 = _dot(x1, wg_ref[...])
    q_raw = _dot(venv * venv, s_ref[...])
    s1 = t0 + g0 * q_raw * (1.0 / 256.0)
    x1_ref[...] = x1
    g0_ref[...] = g0
    s1_ref[...] = s1


def _edge3_body(x1_ref, s1_ref, gg_ref, cut_ref,
                wax_ref, wat_ref, wb_ref, wout_ref, y_ref):
    cut = cut_ref[...]
    x1 = x1_ref[...]
    beta = (1.0 + gg_ref[...] * (1.0 / 16.0)) * RSQRT2
    t1 = beta * s1_ref[...] * RSQRT2
    pre = _dot(x1, wax_ref[...]) + _dot(t1, wat_ref[...])
    x2 = (_dot(_silu(pre), wb_ref[...]) * cut + x1) * RSQRT2
    y_ref[...] = _dot(x2, wout_ref[...]) * cut


def _full(shape):
    return pl.BlockSpec(shape, lambda i: tuple(0 for _ in shape))


def _blk(shape):
    return pl.BlockSpec(shape, lambda i: (i,) + tuple(0 for _ in shape[1:]))


def _node_mm(node_attrs, ws, wr):
    nb = 5
    rb = NN // nb
    return pl.pallas_call(
        _node_mm_body,
        grid=(nb,),
        in_specs=[_blk((rb, D)), _full((D, D)), _full((D, D))],
        out_specs=[_blk((rb, D)), _blk((rb, D))],
        out_shape=[jax.ShapeDtypeStruct((NN, D), jnp.float32)] * 2,
        interpret=_INTERPRET,
    )(node_attrs, ws, wr)


def _edge1(v4, gemb, w1rb, wemb2, wv, r, t):
    nb = EP // BE
    return pl.pallas_call(
        _edge1_body,
        grid=(nb,),
        in_specs=[_blk((BE, 4)), _blk((BE, D)), _full((NR, D)),
                  _full((D, D)), _full((D, DV)), _full((DV, 256)),
                  _full((DV, 256))],
        out_specs=[_blk((BE, 1)), _blk((BE, D)), _blk((BE, DV)),
                   _blk((BE, DV)), _blk((BE, 256))],
        out_shape=[jax.ShapeDtypeStruct((EP, 1), jnp.float32),
                   jax.ShapeDtypeStruct((EP, D), jnp.float32),
                   jax.ShapeDtypeStruct((EP, DV), jnp.float32),
                   jax.ShapeDtypeStruct((EP, DV), jnp.float32),
                   jax.ShapeDtypeStruct((EP, 256), jnp.float32)],
        interpret=_INTERPRET,
    )(v4, gemb, w1rb, wemb2, wv, r, t)


def _edge2(x0, xv, sh, venv, cut, wax, wat, wb, wg, r, s):
    nb = EP // BE
    return pl.pallas_call(
        _edge2_body,
        grid=(nb,),
        in_specs=[_blk((BE, D)), _blk((BE, DV)), _blk((BE, DV)),
                  _blk((BE, 256)), _blk((BE, 1)),
                  _full((D, D)), _full((DV, D)), _full((D, D)),
                  _full((D, DV)), _full((DV, 256)), _full((256, DV))],
        out_specs=[_blk((BE, D)), _blk((BE, DV)), _blk((BE, DV))],
        out_shape=[jax.ShapeDtypeStruct((EP, D), jnp.float32),
                   jax.ShapeDtypeStruct((EP, DV), jnp.float32),
                   jax.ShapeDtypeStruct((EP, DV), jnp.float32)],
        interpret=_INTERPRET,
    )(x0, xv, sh, venv, cut, wax, wat, wb, wg, r, s)


def _edge3(x1, s1, gg, cut, wax, wat, wb, wout):
    nb = EP // BE
    return pl.pallas_call(
        _edge3_body,
        grid=(nb,),
        in_specs=[_blk((BE, D)), _blk((BE, DV)), _blk((BE, DV)),
                  _blk((BE, 1)),
                  _full((D, D)), _full((DV, D)), _full((D, D)),
                  _full((D, 1))],
        out_specs=_blk((BE, 1)),
        out_shape=jax.ShapeDtypeStruct((EP, 1), jnp.float32),
        interpret=_INTERPRET,
    )(x1, s1, gg, cut, wax, wat, wb, wout)


def kernel(node_attrs, vectors, senders, receivers, W_emb1, W_emb2, W_v,
           W0a, W0b, Wg0, W1a, W1b, Wg1, W_out):
    del Wg1
    w1rb = W_emb1[:NR]
    ws = W_emb1[NR:NR + D]
    wr = W_emb1[NR + D:]
    w0ax, w0at = W0a[:D], W0a[D:]
    w1ax, w1at = W1a[:D], W1a[D:]
    r = jnp.asarray(_R_NP)
    t = jnp.asarray(_T_NP)
    s = jnp.asarray(_S_NP)

    pad = EP - E
    pad_ids = NN + (jnp.arange(pad, dtype=jnp.int32) % (NE - NN))
    senders_p = jnp.concatenate([senders, pad_ids])
    receivers_p = jnp.concatenate([receivers, pad_ids])
    v4 = jnp.pad(vectors, ((0, pad), (0, 1)))

    a_s, a_r = _node_mm(node_attrs, ws, wr)
    a_s = jnp.pad(a_s, ((0, NE - NN), (0, 0)))
    a_r = jnp.pad(a_r, ((0, NE - NN), (0, 0)))

    gemb = a_s[senders_p] + a_r[receivers_p]
    cut, x0, xv, sh, v0 = _edge1(v4, gemb, w1rb, W_emb2, W_v, r, t)
    m_raw = jax.ops.segment_sum(v0, senders_p, num_segments=NE)
    venv = m_raw[senders_p]
    x1, g0, s1 = _edge2(x0, xv, sh, venv, cut, w0ax, w0at, W0b, Wg0, r, s)
    g_sum = jax.ops.segment_sum(g0, senders_p, num_segments=NE)
    gg = g_sum[senders_p]
    y = _edge3(x1, s1, gg, cut, w1ax, w1at, W1b, W_out)
    return y[:E]


# speedup vs baseline: 20.5776x; 2.3110x over previous
import functools

import numpy as np
import jax
import jax.numpy as jnp
from jax import lax
from jax.experimental import pallas as pl
from jax.experimental.pallas import tpu as pltpu
from jax.experimental.pallas import tpu_sc as plsc

E = 160000
NN = 10000
EP = 163840
NE = 10240
D = 128
DV = 16
NR = 8
BE = 2048
RSQRT2 = float(1.0 / np.sqrt(2.0))
SQRT2 = float(np.sqrt(2.0))
PI = float(np.pi)

_INTERPRET = False


def _expand_mats():
    R = np.zeros((16, 256), np.float32)
    T = np.zeros((16, 256), np.float32)
    S = np.zeros((256, 16), np.float32)
    for i in range(16):
        for c in range(16):
            R[i, i * 16 + c] = 1.0
            T[c, i * 16 + c] = 1.0
            S[i * 16 + c, c] = 1.0
    return R, T, S


_R_NP, _T_NP, _S_NP = _expand_mats()


def _dot(a, b):
    return jnp.dot(a, b, preferred_element_type=jnp.float32)


def _silu(x):
    return x * jax.nn.sigmoid(x)


def _node_mm_body(x_ref, ws_ref, wr_ref, as_ref, ar_ref):
    x = x_ref[...]
    as_ref[...] = _dot(x, ws_ref[...])
    ar_ref[...] = _dot(x, wr_ref[...])


def _edge1_body(v4_ref, g_ref, w1rb_ref, wemb2_ref, wv_ref, r_ref, t_ref,
                cut_ref, x0_ref, xv_ref, sh_ref, v0_ref):
    v = v4_ref[...]
    d = jnp.sqrt(jnp.sum(v * v, axis=1, keepdims=True))
    xb = jnp.clip(d, 1e-6, 1.0)
    ns = lax.broadcasted_iota(jnp.int32, (v.shape[0], NR), 1).astype(jnp.float32) + 1.0
    rb = SQRT2 * jnp.sin(ns * (PI * xb)) / xb
    xe = jnp.clip(d, 0.0, 1.0)
    x2 = xe * xe
    x3 = x2 * xe
    x6 = x3 * x3
    cut = 1.0 - 28.0 * x6 + 48.0 * x6 * xe - 21.0 * x6 * x2
    u = v / jnp.maximum(d, 1e-6)
    ux, uy, uz = u[:, 0:1], u[:, 1:2], u[:, 2:3]
    one = jnp.ones_like(ux)
    sh = jnp.concatenate([
        one, ux, uy, uz, ux * uy, uy * uz, 3.0 * uz * uz - 1.0, ux * uz,
        ux * ux - uy * uy, uy * (3.0 * ux * ux - uy * uy), ux * uy * uz,
        uy * (5.0 * uz * uz - 1.0), uz * (5.0 * uz * uz - 3.0),
        ux * (5.0 * uz * uz - 1.0), uz * (ux * ux - uy * uy),
        ux * (ux * ux - 3.0 * uy * uy)], axis=1)
    h = g_ref[...] + _dot(rb, w1rb_ref[...])
    x0 = _silu(h)
    x0 = _dot(x0, wemb2_ref[...]) * cut
    xv = _dot(x0, wv_ref[...])
    v0 = _dot(sh, r_ref[...]) * _dot(xv, t_ref[...])
    cut_ref[...] = cut
    x0_ref[...] = x0
    xv_ref[...] = xv
    sh_ref[...] = sh
    v0_ref[...] = v0


def _edge2_body(x0_ref, xv_ref, sh_ref, venv_ref, cut_ref,
                wax_ref, wat_ref, wb_ref, wg_ref, r_ref, s_ref,
                x1_ref, g0_ref, s1_ref):
    venv = venv_ref[...]
    cut = cut_ref[...]
    x0 = x0_ref[...]
    shr = _dot(sh_ref[...], r_ref[...])
    w_raw = _dot(shr * venv, s_ref[...])
    t0 = xv_ref[...] * w_raw * (1.0 / 16.0)
    pre = _dot(x0, wax_ref[...]) + _dot(t0, wat_ref[...])
    x1 = (_dot(_silu(pre), wb_ref[...]) * cut + x0) * RSQRT2
    g0 = _dot(x1, wg_ref[...])
    q_raw = _dot(venv * venv, s_ref[...])
    s1 = t0 + g0 * q_raw * (1.0 / 256.0)
    x1_ref[...] = x1
    g0_ref[...] = g0
    s1_ref[...] = s1


def _edge3_body(x1_ref, s1_ref, gg_ref, cut_ref,
                wax_ref, wat_ref, wb_ref, wout_ref, y_ref):
    cut = cut_ref[...]
    x1 = x1_ref[...]
    beta = (1.0 + gg_ref[...] * (1.0 / 16.0)) * RSQRT2
    t1 = beta * s1_ref[...] * RSQRT2
    pre = _dot(x1, wax_ref[...]) + _dot(t1, wat_ref[...])
    x2 = (_dot(_silu(pre), wb_ref[...]) * cut + x1) * RSQRT2
    y_ref[...] = _dot(x2, wout_ref[...]) * cut


def _full(shape):
    return pl.BlockSpec(shape, lambda i: tuple(0 for _ in shape))


def _blk(shape):
    return pl.BlockSpec(shape, lambda i: (i,) + tuple(0 for _ in shape[1:]))


def _node_mm(node_attrs, ws, wr):
    nb = 5
    rb = NN // nb
    return pl.pallas_call(
        _node_mm_body,
        grid=(nb,),
        in_specs=[_blk((rb, D)), _full((D, D)), _full((D, D))],
        out_specs=[_blk((rb, D)), _blk((rb, D))],
        out_shape=[jax.ShapeDtypeStruct((NN, D), jnp.float32)] * 2,
        interpret=_INTERPRET,
    )(node_attrs, ws, wr)


def _edge1(v4, gemb, w1rb, wemb2, wv, r, t):
    nb = EP // BE
    return pl.pallas_call(
        _edge1_body,
        grid=(nb,),
        in_specs=[_blk((BE, 4)), _blk((BE, D)), _full((NR, D)),
                  _full((D, D)), _full((D, DV)), _full((DV, 256)),
                  _full((DV, 256))],
        out_specs=[_blk((BE, 1)), _blk((BE, D)), _blk((BE, DV)),
                   _blk((BE, DV)), _blk((BE, 256))],
        out_shape=[jax.ShapeDtypeStruct((EP, 1), jnp.float32),
                   jax.ShapeDtypeStruct((EP, D), jnp.float32),
                   jax.ShapeDtypeStruct((EP, DV), jnp.float32),
                   jax.ShapeDtypeStruct((EP, DV), jnp.float32),
                   jax.ShapeDtypeStruct((EP, 256), jnp.float32)],
        interpret=_INTERPRET,
    )(v4, gemb, w1rb, wemb2, wv, r, t)


def _edge2(x0, xv, sh, venv, cut, wax, wat, wb, wg, r, s):
    nb = EP // BE
    return pl.pallas_call(
        _edge2_body,
        grid=(nb,),
        in_specs=[_blk((BE, D)), _blk((BE, DV)), _blk((BE, DV)),
                  _blk((BE, 256)), _blk((BE, 1)),
                  _full((D, D)), _full((DV, D)), _full((D, D)),
                  _full((D, DV)), _full((DV, 256)), _full((256, DV))],
        out_specs=[_blk((BE, D)), _blk((BE, DV)), _blk((BE, DV))],
        out_shape=[jax.ShapeDtypeStruct((EP, D), jnp.float32),
                   jax.ShapeDtypeStruct((EP, DV), jnp.float32),
                   jax.ShapeDtypeStruct((EP, DV), jnp.float32)],
        interpret=_INTERPRET,
    )(x0, xv, sh, venv, cut, wax, wat, wb, wg, r, s)


def _edge3(x1, s1, gg, cut, wax, wat, wb, wout):
    nb = EP // BE
    return pl.pallas_call(
        _edge3_body,
        grid=(nb,),
        in_specs=[_blk((BE, D)), _blk((BE, DV)), _blk((BE, DV)),
                  _blk((BE, 1)),
                  _full((D, D)), _full((DV, D)), _full((D, D)),
                  _full((D, 1))],
        out_specs=_blk((BE, 1)),
        out_shape=jax.ShapeDtypeStruct((EP, 1), jnp.float32),
        interpret=_INTERPRET,
    )(x1, s1, gg, cut, wax, wat, wb, wout)


NWORK = 32
EW = EP // NWORK
CH = 128
NCHW = EW // CH


def _sc_mesh():
    return plsc.VectorSubcoreMesh(core_axis_name="c", subcore_axis_name="s",
                                  num_cores=2, num_subcores=16)


def _sc_gather_add(table_a, table_b, idx_a, idx_b, width):
    @functools.partial(
        pl.kernel, mesh=_sc_mesh(),
        out_type=jax.ShapeDtypeStruct((EP, width), jnp.float32),
        scratch_types=[pltpu.VMEM((CH,), jnp.int32),
                       pltpu.VMEM((CH,), jnp.int32),
                       pltpu.VMEM((CH, width), jnp.float32),
                       pltpu.VMEM((CH, width), jnp.float32),
                       pltpu.SemaphoreType.DMA,
                       pltpu.SemaphoreType.DMA],
    )
    def k(ta, tb, ia, ib, out, iva, ivb, ra, rb, sema, semb):
        wid = lax.axis_index("s") * 2 + lax.axis_index("c")

        @pl.loop(0, NCHW)
        def _chunk(j):
            base = wid * EW + j * CH
            pltpu.sync_copy(ia.at[pl.ds(base, CH)], iva)
            pltpu.sync_copy(ib.at[pl.ds(base, CH)], ivb)
            cpa = pltpu.async_copy(ta.at[iva], ra, sema)
            cpb = pltpu.async_copy(tb.at[ivb], rb, semb)
            cpa.wait()
            cpb.wait()

            @pl.loop(0, CH)
            def _row(r):
                for cc in range(width // 16):
                    sl = pl.ds(cc * 16, 16)
                    ra[r, sl] = ra[r, sl] + rb[r, sl]

            pltpu.sync_copy(ra, out.at[pl.ds(base, CH)])

    return k(table_a, table_b, idx_a, idx_b)


def _sc_gather(table, idx, width):
    @functools.partial(
        pl.kernel, mesh=_sc_mesh(),
        out_type=jax.ShapeDtypeStruct((EP, width), jnp.float32),
        scratch_types=[pltpu.VMEM((CH,), jnp.int32),
                       pltpu.VMEM((CH, width), jnp.float32),
                       pltpu.SemaphoreType.DMA],
    )
    def k(t, ia, out, iv, rows, sem):
        wid = lax.axis_index("s") * 2 + lax.axis_index("c")

        @pl.loop(0, NCHW)
        def _chunk(j):
            base = wid * EW + j * CH
            pltpu.sync_copy(ia.at[pl.ds(base, CH)], iv)
            pltpu.async_copy(t.at[iv], rows, sem).wait()
            pltpu.sync_copy(rows, out.at[pl.ds(base, CH)])

    return k(table, idx)


def _sc_scatter_v(v0, senders_p, zrows):
    nt = 16
    et = EP // nt
    nct = et // CH
    rows_t = NE // nt

    @functools.partial(
        pl.kernel, mesh=_sc_mesh(),
        out_type=jax.ShapeDtypeStruct((NE, 256), jnp.float32),
        scratch_types=[pltpu.VMEM((CH,), jnp.int32),
                       pltpu.VMEM((CH, 128), jnp.float32),
                       pltpu.VMEM_SHARED((NE, 128), jnp.float32)],
    )
    def k(v_hbm, s_hbm, z_hbm, out, iv, vv, shared):
        c = lax.axis_index("c")
        s = lax.axis_index("s")
        pltpu.sync_copy(z_hbm, shared.at[pl.ds(s * rows_t, rows_t)])
        plsc.subcore_barrier()

        @pl.loop(0, nct)
        def _chunk(j):
            base = s * et + j * CH
            pltpu.sync_copy(s_hbm.at[pl.ds(base, CH)], iv)
            pltpu.sync_copy(v_hbm.at[pl.ds(base, CH), pl.ds(c * 128, 128)], vv)
            pltpu.sync_copy(vv, shared.at[iv], add=True)

        plsc.subcore_barrier()
        pltpu.sync_copy(shared.at[pl.ds(s * rows_t, rows_t)],
                        out.at[pl.ds(s * rows_t, rows_t), pl.ds(c * 128, 128)])

    return k(v0, senders_p, zrows)


def _sc_scatter_gather_g(g0, senders_p, zrows16):
    nt = 16
    et = EP // nt
    nct = et // CH
    eh = EP // 2
    etg = eh // nt
    nctg = etg // CH
    rows_t = NE // nt

    @functools.partial(
        pl.kernel, mesh=_sc_mesh(),
        out_type=jax.ShapeDtypeStruct((EP, 16), jnp.float32),
        scratch_types=[pltpu.VMEM((CH,), jnp.int32),
                       pltpu.VMEM((CH, 16), jnp.float32),
                       pltpu.VMEM_SHARED((NE, 16), jnp.float32)],
    )
    def k(g_hbm, s_hbm, z_hbm, out, iv, gv, shared):
        c = lax.axis_index("c")
        s = lax.axis_index("s")
        pltpu.sync_copy(z_hbm, shared.at[pl.ds(s * rows_t, rows_t)])
        plsc.subcore_barrier()

        @pl.loop(0, nct)
        def _chunk(j):
            base = s * et + j * CH
            pltpu.sync_copy(s_hbm.at[pl.ds(base, CH)], iv)
            pltpu.sync_copy(g_hbm.at[pl.ds(base, CH)], gv)
            pltpu.sync_copy(gv, shared.at[iv], add=True)

        plsc.subcore_barrier()

        @pl.loop(0, nctg)
        def _gchunk(j):
            base = c * eh + s * etg + j * CH
            pltpu.sync_copy(s_hbm.at[pl.ds(base, CH)], iv)
            pltpu.sync_copy(shared.at[iv], gv)
            pltpu.sync_copy(gv, out.at[pl.ds(base, CH)])

    return k(g0, senders_p, zrows16)


def kernel(node_attrs, vectors, senders, receivers, W_emb1, W_emb2, W_v,
           W0a, W0b, Wg0, W1a, W1b, Wg1, W_out):
    del Wg1
    w1rb = W_emb1[:NR]
    ws = W_emb1[NR:NR + D]
    wr = W_emb1[NR + D:]
    w0ax, w0at = W0a[:D], W0a[D:]
    w1ax, w1at = W1a[:D], W1a[D:]
    r = jnp.asarray(_R_NP)
    t = jnp.asarray(_T_NP)
    s = jnp.asarray(_S_NP)

    pad = EP - E
    pad_ids = NN + (jnp.arange(pad, dtype=jnp.int32) % (NE - NN))
    senders_p = jnp.concatenate([senders, pad_ids])
    receivers_p = jnp.concatenate([receivers, pad_ids])
    v4 = jnp.pad(vectors, ((0, pad), (0, 1)))

    a_s, a_r = _node_mm(node_attrs, ws, wr)
    a_s = jnp.pad(a_s, ((0, NE - NN), (0, 0)))
    a_r = jnp.pad(a_r, ((0, NE - NN), (0, 0)))

    zrows = jnp.zeros((NE // 16, 128), jnp.float32)
    zrows16 = jnp.zeros((NE // 16, 16), jnp.float32)

    gemb = _sc_gather_add(a_s, a_r, senders_p, receivers_p, D)
    cut, x0, xv, sh, v0 = _edge1(v4, gemb, w1rb, W_emb2, W_v, r, t)
    m_raw = _sc_scatter_v(v0, senders_p, zrows)
    venv = _sc_gather(m_raw, senders_p, 256)
    x1, g0, s1 = _edge2(x0, xv, sh, venv, cut, w0ax, w0at, W0b, Wg0, r, s)
    gg = _sc_scatter_gather_g(g0, senders_p, zrows16)
    y = _edge3(x1, s1, gg, cut, w1ax, w1at, W1b, W_out)
    return y[:E]


# speedup vs baseline: 30.1551x; 1.4654x over previous
import functools

import numpy as np
import jax
import jax.numpy as jnp
from jax import lax
from jax.experimental import pallas as pl
from jax.experimental.pallas import tpu as pltpu
from jax.experimental.pallas import tpu_sc as plsc

E = 160000
NN = 10000
EP = 163840
NE = 10240
D = 128
DV = 16
NR = 8
BE = 2048
RSQRT2 = float(1.0 / np.sqrt(2.0))
SQRT2 = float(np.sqrt(2.0))
PI = float(np.pi)

_INTERPRET = False


def _expand_mats():
    R = np.zeros((16, 256), np.float32)
    T = np.zeros((16, 256), np.float32)
    S = np.zeros((256, 16), np.float32)
    for i in range(16):
        for c in range(16):
            R[i, i * 16 + c] = 1.0
            T[c, i * 16 + c] = 1.0
            S[i * 16 + c, c] = 1.0
    return R, T, S


_R_NP, _T_NP, _S_NP = _expand_mats()


def _dot(a, b):
    return jnp.dot(a, b, preferred_element_type=jnp.float32)


def _silu(x):
    return x * jax.nn.sigmoid(x)


def _node_mm_body(x_ref, ws_ref, wr_ref, as_ref, ar_ref):
    x = x_ref[...]
    as_ref[...] = _dot(x, ws_ref[...])
    ar_ref[...] = _dot(x, wr_ref[...])


def _dot_t(a_t, b):
    return lax.dot_general(a_t, b, (((0,), (0,)), ((), ())),
                           preferred_element_type=jnp.float32)


def _edge1_body(vt_ref, g_ref, w1rb_ref, wemb2_ref, wv_ref, r_ref, t_ref,
                out_ref):
    vx = vt_ref[0:1, :]
    vy = vt_ref[1:2, :]
    vz = vt_ref[2:3, :]
    d = jnp.sqrt(vx * vx + vy * vy + vz * vz)
    xb = jnp.clip(d, 1e-6, 1.0)
    xbinv = 1.0 / xb
    sa = jnp.sin(PI * xb)
    ca = jnp.cos(PI * xb)
    s1, c1 = sa, ca
    s2, c2 = 2.0 * s1 * c1, 1.0 - 2.0 * s1 * s1
    s3, c3 = s2 * c1 + c2 * s1, c2 * c1 - s2 * s1
    s4, c4 = 2.0 * s2 * c2, 1.0 - 2.0 * s2 * s2
    s5 = s4 * c1 + c4 * s1
    s6, c6 = 2.0 * s3 * c3, 1.0 - 2.0 * s3 * s3
    s7 = s6 * c1 + c6 * s1
    s8 = 2.0 * s4 * c4
    rbt = jnp.concatenate([s1, s2, s3, s4, s5, s6, s7, s8],
                          axis=0) * (SQRT2 * xbinv)
    xe = jnp.clip(d, 0.0, 1.0)
    x2 = xe * xe
    x6 = x2 * x2 * x2
    cut_row = 1.0 - 28.0 * x6 + 48.0 * x6 * xe - 21.0 * x6 * x2
    dinv = 1.0 / jnp.maximum(d, 1e-6)
    ux, uy, uz = vx * dinv, vy * dinv, vz * dinv
    one = jnp.ones_like(ux)
    sht = jnp.concatenate([
        one, ux, uy, uz, ux * uy, uy * uz, 3.0 * uz * uz - 1.0, ux * uz,
        ux * ux - uy * uy, uy * (3.0 * ux * ux - uy * uy), ux * uy * uz,
        uy * (5.0 * uz * uz - 1.0), uz * (5.0 * uz * uz - 3.0),
        ux * (5.0 * uz * uz - 1.0), uz * (ux * ux - uy * uy),
        ux * (ux * ux - 3.0 * uy * uy)], axis=0)
    cut = jnp.transpose(cut_row)
    h = g_ref[...] + _dot_t(rbt, w1rb_ref[...])
    x0 = _silu(h)
    x0 = _dot(x0, wemb2_ref[...]) * cut
    xv = _dot(x0, wv_ref[...])
    v0 = _dot_t(sht, r_ref[...]) * _dot(xv, t_ref[...])
    out_ref[:, 0:D] = x0
    out_ref[:, D:D + 16] = xv
    out_ref[:, D + 16:D + 32] = jnp.transpose(sht)
    out_ref[:, D + 32:D + 33] = cut
    out_ref[:, 256:512] = v0


def _edge2_body(in1_ref, venv_ref,
                wax_ref, wat_ref, wb_ref, wg_ref, r_ref, s_ref,
                out_ref, g0_ref):
    in1 = in1_ref[...]
    x0 = in1[:, 0:D]
    xv = in1[:, D:D + 16]
    sh = in1[:, D + 16:D + 32]
    cut = in1[:, D + 32:D + 33]
    venv = venv_ref[...]
    shr = _dot(sh, r_ref[...])
    w_raw = _dot(shr * venv, s_ref[...])
    t0 = xv * w_raw * (1.0 / 16.0)
    pre = _dot(x0, wax_ref[...]) + _dot(t0, wat_ref[...])
    x1 = (_dot(_silu(pre), wb_ref[...]) * cut + x0) * RSQRT2
    g0 = _dot(x1, wg_ref[...])
    q_raw = _dot(venv * venv, s_ref[...])
    s1 = t0 + g0 * q_raw * (1.0 / 256.0)
    out_ref[:, 0:D] = x1
    out_ref[:, D:D + 16] = g0
    out_ref[:, D + 16:D + 32] = s1
    out_ref[:, D + 32:D + 33] = cut
    g0_ref[...] = g0


def _edge3_body(in2_ref, gg_ref,
                wax_ref, wat_ref, wb_ref, wout_ref, y_ref):
    in2 = in2_ref[...]
    x1 = in2[:, 0:D]
    s1 = in2[:, D + 16:D + 32]
    cut = in2[:, D + 32:D + 33]
    beta = (1.0 + gg_ref[...] * (1.0 / 16.0)) * RSQRT2
    t1 = beta * s1 * RSQRT2
    pre = _dot(x1, wax_ref[...]) + _dot(t1, wat_ref[...])
    x2 = (_dot(_silu(pre), wb_ref[...]) * cut + x1) * RSQRT2
    y_ref[...] = _dot(x2, wout_ref[...]) * cut


def _full(shape):
    return pl.BlockSpec(shape, lambda i: tuple(0 for _ in shape))


def _blk(shape):
    return pl.BlockSpec(shape, lambda i: (i,) + tuple(0 for _ in shape[1:]))


def _node_mm(node_attrs, ws, wr):
    nb = 5
    rb = NN // nb
    return pl.pallas_call(
        _node_mm_body,
        grid=(nb,),
        in_specs=[_blk((rb, D)), _full((D, D)), _full((D, D))],
        out_specs=[_blk((rb, D)), _blk((rb, D))],
        out_shape=[jax.ShapeDtypeStruct((NN, D), jnp.float32)] * 2,
        interpret=_INTERPRET,
    )(node_attrs, ws, wr)


def _edge1(vt, gemb, w1rb, wemb2, wv, r, t):
    nb = EP // BE
    return pl.pallas_call(
        _edge1_body,
        grid=(nb,),
        in_specs=[pl.BlockSpec((3, BE), lambda i: (0, i)), _blk((BE, D)),
                  _full((NR, D)), _full((D, D)), _full((D, DV)),
                  _full((DV, 256)), _full((DV, 256))],
        out_specs=_blk((BE, 512)),
        out_shape=jax.ShapeDtypeStruct((EP, 512), jnp.float32),
        interpret=_INTERPRET,
    )(vt, gemb, w1rb, wemb2, wv, r, t)


def _edge2(out1, venv, wax, wat, wb, wg, r, s):
    nb = EP // BE
    return pl.pallas_call(
        _edge2_body,
        grid=(nb,),
        in_specs=[pl.BlockSpec((BE, 256), lambda i: (i, 0)), _blk((BE, 256)),
                  _full((D, D)), _full((DV, D)), _full((D, D)),
                  _full((D, DV)), _full((DV, 256)), _full((256, DV))],
        out_specs=[_blk((BE, 256)), _blk((BE, DV))],
        out_shape=[jax.ShapeDtypeStruct((EP, 256), jnp.float32),
                   jax.ShapeDtypeStruct((EP, DV), jnp.float32)],
        interpret=_INTERPRET,
    )(out1, venv, wax, wat, wb, wg, r, s)


def _edge3(out2, gg, wax, wat, wb, wout):
    nb = EP // BE
    return pl.pallas_call(
        _edge3_body,
        grid=(nb,),
        in_specs=[_blk((BE, 256)), _blk((BE, DV)),
                  _full((D, D)), _full((DV, D)), _full((D, D)),
                  _full((D, 1))],
        out_specs=_blk((BE, 1)),
        out_shape=jax.ShapeDtypeStruct((EP, 1), jnp.float32),
        interpret=_INTERPRET,
    )(out2, gg, wax, wat, wb, wout)


NWORK = 32
EW = EP // NWORK
CH = 128
NCHW = EW // CH


def _sc_mesh():
    return plsc.VectorSubcoreMesh(core_axis_name="c", subcore_axis_name="s",
                                  num_cores=2, num_subcores=16)


def _sc_gather_add(table_a, table_b, idx_a, idx_b, width):
    @functools.partial(
        pl.kernel, mesh=_sc_mesh(),
        out_type=jax.ShapeDtypeStruct((EP, width), jnp.float32),
        scratch_types=[pltpu.VMEM((CH,), jnp.int32),
                       pltpu.VMEM((CH,), jnp.int32),
                       pltpu.VMEM((CH, width), jnp.float32),
                       pltpu.VMEM((CH, width), jnp.float32),
                       pltpu.SemaphoreType.DMA,
                       pltpu.SemaphoreType.DMA],
    )
    def k(ta, tb, ia, ib, out, iva, ivb, ra, rb, sema, semb):
        wid = lax.axis_index("s") * 2 + lax.axis_index("c")

        @pl.loop(0, NCHW)
        def _chunk(j):
            base = wid * EW + j * CH
            pltpu.sync_copy(ia.at[pl.ds(base, CH)], iva)
            pltpu.sync_copy(ib.at[pl.ds(base, CH)], ivb)
            cpa = pltpu.async_copy(ta.at[iva], ra, sema)
            cpb = pltpu.async_copy(tb.at[ivb], rb, semb)
            cpa.wait()
            cpb.wait()

            @pl.loop(0, CH)
            def _row(r):
                for cc in range(width // 16):
                    sl = pl.ds(cc * 16, 16)
                    ra[r, sl] = ra[r, sl] + rb[r, sl]

            pltpu.sync_copy(ra, out.at[pl.ds(base, CH)])

    return k(table_a, table_b, idx_a, idx_b)


def _sc_gather(table, idx, width):
    @functools.partial(
        pl.kernel, mesh=_sc_mesh(),
        out_type=jax.ShapeDtypeStruct((EP, width), jnp.float32),
        scratch_types=[pltpu.VMEM((CH,), jnp.int32),
                       pltpu.VMEM((CH, width), jnp.float32),
                       pltpu.SemaphoreType.DMA],
    )
    def k(t, ia, out, iv, rows, sem):
        wid = lax.axis_index("s") * 2 + lax.axis_index("c")

        @pl.loop(0, NCHW)
        def _chunk(j):
            base = wid * EW + j * CH
            pltpu.sync_copy(ia.at[pl.ds(base, CH)], iv)
            pltpu.async_copy(t.at[iv], rows, sem).wait()
            pltpu.sync_copy(rows, out.at[pl.ds(base, CH)])

    return k(table, idx)


def _sc_scatter_v(v0, senders_p, zrows):
    nt = 16
    et = EP // nt
    nct = et // CH
    rows_t = NE // nt

    @functools.partial(
        pl.kernel, mesh=_sc_mesh(),
        out_type=jax.ShapeDtypeStruct((NE, 256), jnp.float32),
        scratch_types=[pltpu.VMEM((CH,), jnp.int32),
                       pltpu.VMEM((CH, 128), jnp.float32),
                       pltpu.VMEM_SHARED((NE, 128), jnp.float32)],
    )
    def k(v_hbm, s_hbm, z_hbm, out, iv, vv, shared):
        c = lax.axis_index("c")
        s = lax.axis_index("s")
        pltpu.sync_copy(z_hbm, shared.at[pl.ds(s * rows_t, rows_t)])
        plsc.subcore_barrier()

        @pl.loop(0, nct)
        def _chunk(j):
            base = s * et + j * CH
            pltpu.sync_copy(s_hbm.at[pl.ds(base, CH)], iv)
            pltpu.sync_copy(v_hbm.at[pl.ds(base, CH), pl.ds(256 + c * 128, 128)],
                            vv)
            pltpu.sync_copy(vv, shared.at[iv], add=True)

        plsc.subcore_barrier()
        pltpu.sync_copy(shared.at[pl.ds(s * rows_t, rows_t)],
                        out.at[pl.ds(s * rows_t, rows_t), pl.ds(c * 128, 128)])

    return k(v0, senders_p, zrows)


def _sc_scatter_gather_g(g0, senders_p, zrows16):
    nt = 16
    et = EP // nt
    nct = et // CH
    eh = EP // 2
    etg = eh // nt
    nctg = etg // CH
    rows_t = NE // nt

    @functools.partial(
        pl.kernel, mesh=_sc_mesh(),
        out_type=jax.ShapeDtypeStruct((EP, 16), jnp.float32),
        scratch_types=[pltpu.VMEM((CH,), jnp.int32),
                       pltpu.VMEM((CH, 16), jnp.float32),
                       pltpu.VMEM_SHARED((NE, 16), jnp.float32)],
    )
    def k(g_hbm, s_hbm, z_hbm, out, iv, gv, shared):
        c = lax.axis_index("c")
        s = lax.axis_index("s")
        pltpu.sync_copy(z_hbm, shared.at[pl.ds(s * rows_t, rows_t)])
        plsc.subcore_barrier()

        @pl.loop(0, nct)
        def _chunk(j):
            base = s * et + j * CH
            pltpu.sync_copy(s_hbm.at[pl.ds(base, CH)], iv)
            pltpu.sync_copy(g_hbm.at[pl.ds(base, CH)], gv)
            pltpu.sync_copy(gv, shared.at[iv], add=True)

        plsc.subcore_barrier()

        @pl.loop(0, nctg)
        def _gchunk(j):
            base = c * eh + s * etg + j * CH
            pltpu.sync_copy(s_hbm.at[pl.ds(base, CH)], iv)
            pltpu.sync_copy(shared.at[iv], gv)
            pltpu.sync_copy(gv, out.at[pl.ds(base, CH)])

    return k(g0, senders_p, zrows16)


def kernel(node_attrs, vectors, senders, receivers, W_emb1, W_emb2, W_v,
           W0a, W0b, Wg0, W1a, W1b, Wg1, W_out):
    del Wg1
    w1rb = W_emb1[:NR]
    ws = W_emb1[NR:NR + D]
    wr = W_emb1[NR + D:]
    w0ax, w0at = W0a[:D], W0a[D:]
    w1ax, w1at = W1a[:D], W1a[D:]
    r = jnp.asarray(_R_NP)
    t = jnp.asarray(_T_NP)
    s = jnp.asarray(_S_NP)

    pad = EP - E
    pad_ids = NN + (jnp.arange(pad, dtype=jnp.int32) % (NE - NN))
    senders_p = jnp.concatenate([senders, pad_ids])
    receivers_p = jnp.concatenate([receivers, pad_ids])
    vt = jnp.pad(vectors, ((0, pad), (0, 0))).T

    a_s, a_r = _node_mm(node_attrs, ws, wr)
    a_s = jnp.pad(a_s, ((0, NE - NN), (0, 0)))
    a_r = jnp.pad(a_r, ((0, NE - NN), (0, 0)))

    zrows = jnp.zeros((NE // 16, 128), jnp.float32)
    zrows16 = jnp.zeros((NE // 16, 16), jnp.float32)

    gemb = _sc_gather_add(a_s, a_r, senders_p, receivers_p, D)
    out1 = _edge1(vt, gemb, w1rb, W_emb2, W_v, r, t)
    m_raw = _sc_scatter_v(out1, senders_p, zrows)
    venv = _sc_gather(m_raw, senders_p, 256)
    out2, g0 = _edge2(out1, venv, w0ax, w0at, W0b, Wg0, r, s)
    gg = _sc_scatter_gather_g(g0, senders_p, zrows16)
    y = _edge3(out2, gg, w1ax, w1at, W1b, W_out)
    return y[:E]


# speedup vs baseline: 38.0291x; 1.2611x over previous
import functools

import numpy as np
import jax
import jax.numpy as jnp
from jax import lax
from jax.experimental import pallas as pl
from jax.experimental.pallas import tpu as pltpu
from jax.experimental.pallas import tpu_sc as plsc

E = 160000
NN = 10000
EP = 163840
NE = 10240
D = 128
DV = 16
NR = 8
BE = 2048
RSQRT2 = float(1.0 / np.sqrt(2.0))
SQRT2 = float(np.sqrt(2.0))
PI = float(np.pi)

_INTERPRET = False


def _expand_mats():
    R = np.zeros((16, 256), np.float32)
    T = np.zeros((16, 256), np.float32)
    S = np.zeros((256, 16), np.float32)
    for i in range(16):
        for c in range(16):
            R[i, i * 16 + c] = 1.0
            T[c, i * 16 + c] = 1.0
            S[i * 16 + c, c] = 1.0
    return R, T, S


_R_NP, _T_NP, _S_NP = _expand_mats()


def _dot(a, b):
    return jnp.dot(a, b, preferred_element_type=jnp.float32)


def _silu(x):
    return x * jax.nn.sigmoid(x)


def _node_mm_body(x_ref, ws_ref, wr_ref, as_ref, ar_ref):
    x = x_ref[...]
    as_ref[...] = _dot(x, ws_ref[...])
    ar_ref[...] = _dot(x, wr_ref[...])


def _dot_t(a_t, b):
    return lax.dot_general(a_t, b, (((0,), (0,)), ((), ())),
                           preferred_element_type=jnp.float32)


def _edge1_body(vt_ref, g_ref, w1rb_ref, wemb2_ref, wv_ref, r_ref, t_ref,
                out_ref):
    vx = vt_ref[0:1, :]
    vy = vt_ref[1:2, :]
    vz = vt_ref[2:3, :]
    d = jnp.sqrt(vx * vx + vy * vy + vz * vz)
    xb = jnp.clip(d, 1e-6, 1.0)
    xbinv = 1.0 / xb
    sa = jnp.sin(PI * xb)
    ca = jnp.cos(PI * xb)
    s1, c1 = sa, ca
    s2, c2 = 2.0 * s1 * c1, 1.0 - 2.0 * s1 * s1
    s3, c3 = s2 * c1 + c2 * s1, c2 * c1 - s2 * s1
    s4, c4 = 2.0 * s2 * c2, 1.0 - 2.0 * s2 * s2
    s5 = s4 * c1 + c4 * s1
    s6, c6 = 2.0 * s3 * c3, 1.0 - 2.0 * s3 * s3
    s7 = s6 * c1 + c6 * s1
    s8 = 2.0 * s4 * c4
    rbt = jnp.concatenate([s1, s2, s3, s4, s5, s6, s7, s8],
                          axis=0) * (SQRT2 * xbinv)
    xe = jnp.clip(d, 0.0, 1.0)
    x2 = xe * xe
    x6 = x2 * x2 * x2
    cut_row = 1.0 - 28.0 * x6 + 48.0 * x6 * xe - 21.0 * x6 * x2
    dinv = 1.0 / jnp.maximum(d, 1e-6)
    ux, uy, uz = vx * dinv, vy * dinv, vz * dinv
    one = jnp.ones_like(ux)
    sht = jnp.concatenate([
        one, ux, uy, uz, ux * uy, uy * uz, 3.0 * uz * uz - 1.0, ux * uz,
        ux * ux - uy * uy, uy * (3.0 * ux * ux - uy * uy), ux * uy * uz,
        uy * (5.0 * uz * uz - 1.0), uz * (5.0 * uz * uz - 3.0),
        ux * (5.0 * uz * uz - 1.0), uz * (ux * ux - uy * uy),
        ux * (ux * ux - 3.0 * uy * uy)], axis=0)
    cut = jnp.transpose(cut_row)
    h = g_ref[...] + _dot_t(rbt, w1rb_ref[...])
    x0 = _silu(h)
    x0 = _dot(x0, wemb2_ref[...]) * cut
    xv = _dot(x0, wv_ref[...])
    v0 = _dot_t(sht, r_ref[...]) * _dot(xv, t_ref[...])
    out_ref[:, 0:D] = x0
    out_ref[:, D:D + 16] = xv
    out_ref[:, D + 16:D + 32] = jnp.transpose(sht)
    out_ref[:, D + 32:D + 33] = cut
    out_ref[:, 256:512] = v0


def _edge2_body(in1_ref, venv_ref,
                wax_ref, wat_ref, wb_ref, wg_ref, r_ref, s_ref,
                out_ref, g0_ref):
    in1 = in1_ref[...]
    x0 = in1[:, 0:D]
    xv = in1[:, D:D + 16]
    sh = in1[:, D + 16:D + 32]
    cut = in1[:, D + 32:D + 33]
    venv = venv_ref[...]
    shr = _dot(sh, r_ref[...])
    w_raw = _dot(shr * venv, s_ref[...])
    t0 = xv * w_raw * (1.0 / 16.0)
    pre = _dot(x0, wax_ref[...]) + _dot(t0, wat_ref[...])
    x1 = (_dot(_silu(pre), wb_ref[...]) * cut + x0) * RSQRT2
    g0 = _dot(x1, wg_ref[...])
    q_raw = _dot(venv * venv, s_ref[...])
    s1 = t0 + g0 * q_raw * (1.0 / 256.0)
    out_ref[:, 0:D] = x1
    out_ref[:, D:D + 16] = g0
    out_ref[:, D + 16:D + 32] = s1
    out_ref[:, D + 32:D + 33] = cut
    g0_ref[...] = g0


def _edge3_body(in2_ref, gg_ref,
                wax_ref, wat_ref, wb_ref, wout_ref, y_ref):
    in2 = in2_ref[...]
    x1 = in2[:, 0:D]
    s1 = in2[:, D + 16:D + 32]
    cut = in2[:, D + 32:D + 33]
    beta = (1.0 + gg_ref[...] * (1.0 / 16.0)) * RSQRT2
    t1 = beta * s1 * RSQRT2
    pre = _dot(x1, wax_ref[...]) + _dot(t1, wat_ref[...])
    x2 = (_dot(_silu(pre), wb_ref[...]) * cut + x1) * RSQRT2
    y_ref[...] = _dot(x2, wout_ref[...]) * cut


def _full(shape):
    return pl.BlockSpec(shape, lambda i: tuple(0 for _ in shape))


def _blk(shape):
    return pl.BlockSpec(shape, lambda i: (i,) + tuple(0 for _ in shape[1:]))


def _node_mm(node_attrs, ws, wr):
    nb = 5
    rb = NN // nb
    return pl.pallas_call(
        _node_mm_body,
        grid=(nb,),
        in_specs=[_blk((rb, D)), _full((D, D)), _full((D, D))],
        out_specs=[_blk((rb, D)), _blk((rb, D))],
        out_shape=[jax.ShapeDtypeStruct((NN, D), jnp.float32)] * 2,
        interpret=_INTERPRET,
    )(node_attrs, ws, wr)


def _edge1(vt, gemb, w1rb, wemb2, wv, r, t):
    nb = EP // BE
    return pl.pallas_call(
        _edge1_body,
        grid=(nb,),
        in_specs=[pl.BlockSpec((3, BE), lambda i: (0, i)), _blk((BE, D)),
                  _full((NR, D)), _full((D, D)), _full((D, DV)),
                  _full((DV, 256)), _full((DV, 256))],
        out_specs=_blk((BE, 512)),
        out_shape=jax.ShapeDtypeStruct((EP, 512), jnp.float32),
        interpret=_INTERPRET,
    )(vt, gemb, w1rb, wemb2, wv, r, t)


def _edge2(out1, venv, wax, wat, wb, wg, r, s):
    nb = EP // BE
    return pl.pallas_call(
        _edge2_body,
        grid=(nb,),
        in_specs=[pl.BlockSpec((BE, 256), lambda i: (i, 0)), _blk((BE, 256)),
                  _full((D, D)), _full((DV, D)), _full((D, D)),
                  _full((D, DV)), _full((DV, 256)), _full((256, DV))],
        out_specs=[_blk((BE, 256)), _blk((BE, DV))],
        out_shape=[jax.ShapeDtypeStruct((EP, 256), jnp.float32),
                   jax.ShapeDtypeStruct((EP, DV), jnp.float32)],
        interpret=_INTERPRET,
    )(out1, venv, wax, wat, wb, wg, r, s)


def _edge3(out2, gg, wax, wat, wb, wout):
    nb = EP // BE
    return pl.pallas_call(
        _edge3_body,
        grid=(nb,),
        in_specs=[_blk((BE, 256)), _blk((BE, DV)),
                  _full((D, D)), _full((DV, D)), _full((D, D)),
                  _full((D, 1))],
        out_specs=_blk((BE, 1)),
        out_shape=jax.ShapeDtypeStruct((EP, 1), jnp.float32),
        interpret=_INTERPRET,
    )(out2, gg, wax, wat, wb, wout)


NWORK = 32
EW = EP // NWORK
CH = 128
NCHW = EW // CH


def _sc_mesh():
    return plsc.VectorSubcoreMesh(core_axis_name="c", subcore_axis_name="s",
                                  num_cores=2, num_subcores=16)


def _sc_gather_add(table_a, table_b, idx_a, idx_b, width):
    @functools.partial(
        pl.kernel, mesh=_sc_mesh(),
        out_type=jax.ShapeDtypeStruct((EP, width), jnp.float32),
        scratch_types=[pltpu.VMEM((EW,), jnp.int32),
                       pltpu.VMEM((EW,), jnp.int32),
                       pltpu.VMEM((CH, width), jnp.float32),
                       pltpu.VMEM((CH, width), jnp.float32),
                       pltpu.VMEM((CH, width), jnp.float32),
                       pltpu.VMEM((CH, width), jnp.float32),
                       pltpu.SemaphoreType.DMA,
                       pltpu.SemaphoreType.DMA,
                       pltpu.SemaphoreType.DMA,
                       pltpu.SemaphoreType.DMA],
    )
    def k(ta, tb, ia, ib, out, iva, ivb, ra0, rb0, ra1, rb1,
          sa0, sb0, sa1, sb1):
        wid = lax.axis_index("s") * 2 + lax.axis_index("c")
        base0 = wid * EW
        pltpu.sync_copy(ia.at[pl.ds(base0, EW)], iva)
        pltpu.sync_copy(ib.at[pl.ds(base0, EW)], ivb)
        bufs = ((ra0, rb0, sa0, sb0), (ra1, rb1, sa1, sb1))
        pltpu.async_copy(ta.at[iva.at[pl.ds(0, CH)]], ra0, sa0)
        pltpu.async_copy(tb.at[ivb.at[pl.ds(0, CH)]], rb0, sb0)

        @pl.loop(0, NCHW, step=2)
        def _chunk(j):
            for p in range(2):
                jj = j + p
                ra, rb, sa, sb = bufs[p]
                na, nb, nsa, nsb = bufs[1 - p]
                pltpu.make_async_copy(
                    ta.at[iva.at[pl.ds(jj * CH, CH)]], ra, sa).wait()
                pltpu.make_async_copy(
                    tb.at[ivb.at[pl.ds(jj * CH, CH)]], rb, sb).wait()

                @pl.when(jj + 1 < NCHW)
                def _():
                    nsl = pl.ds((jj + 1) * CH, CH)
                    pltpu.async_copy(ta.at[iva.at[nsl]], na, nsa)
                    pltpu.async_copy(tb.at[ivb.at[nsl]], nb, nsb)

                @pl.loop(0, CH)
                def _row(r):
                    for cc in range(width // 16):
                        sl = pl.ds(cc * 16, 16)
                        ra[r, sl] = ra[r, sl] + rb[r, sl]

                pltpu.sync_copy(ra, out.at[pl.ds(base0 + jj * CH, CH)])

    return k(table_a, table_b, idx_a, idx_b)


def _sc_gather(table, idx, width):
    @functools.partial(
        pl.kernel, mesh=_sc_mesh(),
        out_type=jax.ShapeDtypeStruct((EP, width), jnp.float32),
        scratch_types=[pltpu.VMEM((EW,), jnp.int32),
                       pltpu.VMEM((CH, width), jnp.float32),
                       pltpu.VMEM((CH, width), jnp.float32),
                       pltpu.SemaphoreType.DMA,
                       pltpu.SemaphoreType.DMA],
    )
    def k(t, ia, out, iva, r0, r1, s0, s1):
        wid = lax.axis_index("s") * 2 + lax.axis_index("c")
        base0 = wid * EW
        pltpu.sync_copy(ia.at[pl.ds(base0, EW)], iva)
        bufs = ((r0, s0), (r1, s1))
        pltpu.async_copy(t.at[iva.at[pl.ds(0, CH)]], r0, s0)

        @pl.loop(0, NCHW, step=2)
        def _chunk(j):
            for p in range(2):
                jj = j + p
                r, s = bufs[p]
                nr, ns = bufs[1 - p]
                pltpu.make_async_copy(
                    t.at[iva.at[pl.ds(jj * CH, CH)]], r, s).wait()

                @pl.when(jj + 1 < NCHW)
                def _():
                    pltpu.async_copy(
                        t.at[iva.at[pl.ds((jj + 1) * CH, CH)]], nr, ns)

                pltpu.sync_copy(r, out.at[pl.ds(base0 + jj * CH, CH)])

    return k(table, idx)


def _sc_scatter_v(v0, senders_p, zrows):
    nt = 16
    et = EP // nt
    nct = et // CH
    rows_t = NE // nt

    @functools.partial(
        pl.kernel, mesh=_sc_mesh(),
        out_type=jax.ShapeDtypeStruct((NE, 256), jnp.float32),
        scratch_types=[pltpu.VMEM((CH,), jnp.int32),
                       pltpu.VMEM((CH,), jnp.int32),
                       pltpu.VMEM((CH, 128), jnp.float32),
                       pltpu.VMEM((CH, 128), jnp.float32),
                       pltpu.SemaphoreType.DMA,
                       pltpu.SemaphoreType.DMA,
                       pltpu.VMEM_SHARED((NE, 128), jnp.float32)],
    )
    def k(v_hbm, s_hbm, z_hbm, out, iv0, iv1, vv0, vv1, s0, s1, shared):
        c = lax.axis_index("c")
        s = lax.axis_index("s")
        col = pl.ds(256 + c * 128, 128)
        pltpu.sync_copy(z_hbm, shared.at[pl.ds(s * rows_t, rows_t)])
        plsc.subcore_barrier()
        bufs = ((vv0, iv0, s0), (vv1, iv1, s1))
        pltpu.async_copy(s_hbm.at[pl.ds(s * et, CH)], iv0, s0)
        pltpu.async_copy(v_hbm.at[pl.ds(s * et, CH), col], vv0, s0)

        @pl.loop(0, nct, step=2)
        def _chunk(j):
            for p in range(2):
                jj = j + p
                vv, iv, sm = bufs[p]
                nv, niv, nsm = bufs[1 - p]
                pltpu.make_async_copy(
                    s_hbm.at[pl.ds(s * et + jj * CH, CH)], iv, sm).wait()
                pltpu.make_async_copy(
                    v_hbm.at[pl.ds(s * et + jj * CH, CH), col], vv, sm).wait()

                @pl.when(jj + 1 < nct)
                def _():
                    nsl = pl.ds(s * et + (jj + 1) * CH, CH)
                    pltpu.async_copy(s_hbm.at[nsl], niv, nsm)
                    pltpu.async_copy(v_hbm.at[nsl, col], nv, nsm)

                pltpu.sync_copy(vv, shared.at[iv], add=True)

        plsc.subcore_barrier()
        pltpu.sync_copy(shared.at[pl.ds(s * rows_t, rows_t)],
                        out.at[pl.ds(s * rows_t, rows_t), pl.ds(c * 128, 128)])

    return k(v0, senders_p, zrows)


def _sc_scatter_gather_g(g0, senders_p, zrows16):
    nt = 16
    et = EP // nt
    nct = et // CH
    eh = EP // 2
    etg = eh // nt
    nctg = etg // CH
    rows_t = NE // nt

    @functools.partial(
        pl.kernel, mesh=_sc_mesh(),
        out_type=jax.ShapeDtypeStruct((EP, 16), jnp.float32),
        scratch_types=[pltpu.VMEM((etg,), jnp.int32),
                       pltpu.VMEM((CH,), jnp.int32),
                       pltpu.VMEM((CH,), jnp.int32),
                       pltpu.VMEM((CH, 16), jnp.float32),
                       pltpu.VMEM((CH, 16), jnp.float32),
                       pltpu.SemaphoreType.DMA,
                       pltpu.SemaphoreType.DMA,
                       pltpu.VMEM_SHARED((NE, 16), jnp.float32)],
    )
    def k(g_hbm, s_hbm, z_hbm, out, ivg, iv0, iv1, gv0, gv1, s0, s1,
          shared):
        c = lax.axis_index("c")
        s = lax.axis_index("s")
        pltpu.sync_copy(z_hbm, shared.at[pl.ds(s * rows_t, rows_t)])
        pltpu.sync_copy(s_hbm.at[pl.ds(c * eh + s * etg, etg)], ivg)
        plsc.subcore_barrier()
        bufs = ((gv0, iv0, s0), (gv1, iv1, s1))
        pltpu.async_copy(s_hbm.at[pl.ds(s * et, CH)], iv0, s0)
        pltpu.async_copy(g_hbm.at[pl.ds(s * et, CH)], gv0, s0)

        @pl.loop(0, nct, step=2)
        def _chunk(j):
            for p in range(2):
                jj = j + p
                gv, iv, sm = bufs[p]
                ng, niv, nsm = bufs[1 - p]
                pltpu.make_async_copy(
                    s_hbm.at[pl.ds(s * et + jj * CH, CH)], iv, sm).wait()
                pltpu.make_async_copy(
                    g_hbm.at[pl.ds(s * et + jj * CH, CH)], gv, sm).wait()

                @pl.when(jj + 1 < nct)
                def _():
                    nsl = pl.ds(s * et + (jj + 1) * CH, CH)
                    pltpu.async_copy(s_hbm.at[nsl], niv, nsm)
                    pltpu.async_copy(g_hbm.at[nsl], ng, nsm)

                pltpu.sync_copy(gv, shared.at[iv], add=True)

        plsc.subcore_barrier()
        gbase = c * eh + s * etg
        pltpu.async_copy(shared.at[ivg.at[pl.ds(0, CH)]], gv0, s0)

        @pl.loop(0, nctg, step=2)
        def _gchunk(j):
            for p in range(2):
                jj = j + p
                gv, _, sm = bufs[p]
                ng, _, nsm = bufs[1 - p]
                pltpu.make_async_copy(
                    shared.at[ivg.at[pl.ds(jj * CH, CH)]], gv, sm).wait()

                @pl.when(jj + 1 < nctg)
                def _():
                    pltpu.async_copy(
                        shared.at[ivg.at[pl.ds((jj + 1) * CH, CH)]], ng, nsm)

                pltpu.sync_copy(gv, out.at[pl.ds(gbase + jj * CH, CH)])

    return k(g0, senders_p, zrows16)


def kernel(node_attrs, vectors, senders, receivers, W_emb1, W_emb2, W_v,
           W0a, W0b, Wg0, W1a, W1b, Wg1, W_out):
    del Wg1
    w1rb = W_emb1[:NR]
    ws = W_emb1[NR:NR + D]
    wr = W_emb1[NR + D:]
    w0ax, w0at = W0a[:D], W0a[D:]
    w1ax, w1at = W1a[:D], W1a[D:]
    r = jnp.asarray(_R_NP)
    t = jnp.asarray(_T_NP)
    s = jnp.asarray(_S_NP)

    pad = EP - E
    pad_ids = NN + (jnp.arange(pad, dtype=jnp.int32) % (NE - NN))
    senders_p = jnp.concatenate([senders, pad_ids])
    receivers_p = jnp.concatenate([receivers, pad_ids])
    vt = jnp.pad(vectors, ((0, pad), (0, 0))).T

    a_s, a_r = _node_mm(node_attrs, ws, wr)
    a_s = jnp.pad(a_s, ((0, NE - NN), (0, 0)))
    a_r = jnp.pad(a_r, ((0, NE - NN), (0, 0)))

    zrows = jnp.zeros((NE // 16, 128), jnp.float32)
    zrows16 = jnp.zeros((NE // 16, 16), jnp.float32)

    gemb = _sc_gather_add(a_s, a_r, senders_p, receivers_p, D)
    out1 = _edge1(vt, gemb, w1rb, W_emb2, W_v, r, t)
    m_raw = _sc_scatter_v(out1, senders_p, zrows)
    venv = _sc_gather(m_raw, senders_p, 256)
    out2, g0 = _edge2(out1, venv, w0ax, w0at, W0b, Wg0, r, s)
    gg = _sc_scatter_gather_g(g0, senders_p, zrows16)
    y = _edge3(out2, gg, w1ax, w1at, W1b, W_out)
    return y[:E]
